# baseline (speedup 1.0000x reference)
import functools

import jax
import jax.numpy as jnp
from jax import lax
from jax.experimental import pallas as pl
from jax.experimental.pallas import tpu as pltpu
from jax.experimental.pallas import tpu_sc as plsc

N = 10000
E = 320000
D = 128
K = 8
OUT_DIM = 128

DH = D // 2
NS = 16
NP = 10240
SLAB = NP // NS
CH = 128
NCHUNK = 2560
EP = NCHUNK * CH
CPT = NCHUNK // NS
TRASH = NP + 64
GROWS = NP + 256
RB = 128
NB = SLAB // RB
YROWS = (K + 1) * NP
NSLOT = 4


def _rsqrt16(d):
    bi = lax.bitcast_convert_type(d, jnp.int32)
    bi = jnp.int32(0x5F3759DF) - lax.shift_right_logical(bi, 1)
    x = lax.bitcast_convert_type(bi, jnp.float32)
    half = d * 0.5
    for _ in range(3):
        x = x * (1.5 - half * x * x)
    return x


def _sc_propagate(fab, pk_edges):
    mesh = plsc.VectorSubcoreMesh(
        core_axis_name="c", subcore_axis_name="s", num_cores=2
    )

    @functools.partial(
        pl.kernel,
        out_type=[
            jax.ShapeDtypeStruct((2 * YROWS, DH), jnp.float32),
            jax.ShapeDtypeStruct((2 * NP,), jnp.float32),
        ],
        mesh=mesh,
        compiler_params=pltpu.CompilerParams(use_tc_tiling_on_sc=False),
        scratch_types=[
            [pltpu.VMEM((CH, DH), jnp.float32) for _ in range(NSLOT)],
            [pltpu.VMEM((2 * CH,), jnp.int32) for _ in range(NSLOT)],
            [pltpu.VMEM((CH,), jnp.int32) for _ in range(NSLOT)],
            [pltpu.VMEM((CH,), jnp.int32) for _ in range(NSLOT)],
            pltpu.VMEM((CH, DH), jnp.float32),
            pltpu.VMEM((CH, DH), jnp.float32),
            pltpu.VMEM((CH,), jnp.float32),
            pltpu.VMEM((SLAB,), jnp.float32),
            pltpu.VMEM((SLAB,), jnp.float32),
            pltpu.VMEM((SLAB,), jnp.float32),
            pltpu.VMEM_SHARED((GROWS, DH), jnp.float32),
            pltpu.VMEM_SHARED((GROWS,), jnp.float32),
            [pltpu.SemaphoreType.DMA for _ in range(NSLOT)],
            [pltpu.SemaphoreType.DMA for _ in range(NSLOT)],
            [pltpu.SemaphoreType.DMA for _ in range(NSLOT)],
            [pltpu.SemaphoreType.DMA for _ in range(NSLOT)],
        ],
    )
    def k(fab_r, pk, Yab, sinv, rows_b, pk_b, idx_r, idx_c, aux_buf, zbuf,
          ones_buf, deg_v, s_v, t_v, g, deg1, sem_i, sem_c, sem_g, sem_s):
        cid = lax.axis_index("c")
        wid = lax.axis_index("s")
        nbase = wid * SLAB
        ybase = cid * YROWS

        def fill16(r, _):
            ones_buf[pl.ds(r * 16, 16)] = jnp.full((16,), 1.0, jnp.float32)
            return 0
        lax.fori_loop(0, CH // 16, fill16, 0)

        def fillz(r, _):
            for j in range(DH // 16):
                zbuf[r, pl.ds(j * 16, 16)] = jnp.zeros((16,), jnp.float32)
            return 0
        lax.fori_loop(0, CH, fillz, 0)

        def fill_slab(r, _):
            s_v[pl.ds(r * 16, 16)] = jnp.full((16,), 1.0, jnp.float32)
            return 0
        lax.fori_loop(0, SLAB // 16, fill_slab, 0)
        pltpu.sync_copy(s_v, deg1.at[pl.ds(nbase, SLAB)])

        def zeroblk(bidx, _):
            pltpu.sync_copy(zbuf, g.at[pl.ds(nbase + bidx * RB, RB), :])
            return 0
        lax.fori_loop(0, NB, zeroblk, 0)
        plsc.subcore_barrier()

        def hist(gq, _):
            for b in range(2):
                j = 2 * gq + b

                @pl.when(gq >= 1)
                def _():
                    pltpu.make_async_copy(
                        ones_buf, deg1.at[idx_c[b]], sem_s[b]
                    ).wait()
                off = (wid + NS * j) * 2 * CH + CH
                pltpu.sync_copy(pk.at[pl.ds(off, CH)], idx_c[b])
                pltpu.async_copy(ones_buf, deg1.at[idx_c[b]], sem_s[b],
                                 add=True)
            return 0
        lax.fori_loop(0, CPT // 2, hist, 0)
        for b in range(2):
            pltpu.make_async_copy(ones_buf, deg1.at[idx_c[b]], sem_s[b]).wait()
        plsc.subcore_barrier()

        pltpu.sync_copy(deg1.at[pl.ds(nbase, SLAB)], deg_v)

        def newton(r, _):
            sl = pl.ds(r * 16, 16)
            d = deg_v[sl]
            x = _rsqrt16(d)
            s_v[sl] = x
            t_v[sl] = x * x
            deg_v[sl] = d * x
            return 0
        lax.fori_loop(0, SLAB // 16, newton, 0)
        pltpu.sync_copy(deg_v, sinv.at[pl.ds(cid * NP + nbase, SLAB)])

        def y0blk(bidx, _):
            base = bidx * RB
            pltpu.sync_copy(
                fab_r.at[pl.ds(cid * NP + nbase + base, RB), :], rows_b[0]
            )

            def srow(r16, _):
                sv16 = s_v[pl.ds(base + r16 * 16, 16)]
                for ri in range(16):
                    sv = jnp.full((16,), sv16[ri], jnp.float32)
                    row = r16 * 16 + ri
                    for j in range(DH // 16):
                        sl = pl.ds(j * 16, 16)
                        rows_b[0][row, sl] = rows_b[0][row, sl] * sv
                return 0
            lax.fori_loop(0, RB // 16, srow, 0)
            pltpu.sync_copy(
                rows_b[0], Yab.at[pl.ds(ybase + nbase + base, RB), :]
            )
            return 0
        lax.fori_loop(0, NB, y0blk, 0)
        plsc.subcore_barrier()

        def hop_body(h, _):
            src_off = ybase + h * NP
            ovec = jnp.full((16,), src_off, jnp.int32)

            def prep(jj, b):
                pltpu.make_async_copy(
                    pk.at[pl.ds(0, 2 * CH)], pk_b[b], sem_i[b]).wait()
                pltpu.make_async_copy(
                    pk.at[pl.ds(0, CH)], idx_c[b], sem_c[b]).wait()

                def unpack(q, _):
                    sl = pl.ds(q * 16, 16)
                    idx_r[b][sl] = pk_b[b][sl] + ovec
                    return 0
                lax.fori_loop(0, CH // 16, unpack, 0)

            def load_pk(jj, b):
                off = (wid + NS * jj) * 2 * CH
                pltpu.async_copy(pk.at[pl.ds(off, 2 * CH)], pk_b[b], sem_i[b])
                pltpu.async_copy(pk.at[pl.ds(off + CH, CH)], idx_c[b],
                                 sem_c[b])

            load_pk(0, 0)
            prep(0, 0)
            pltpu.async_copy(Yab.at[idx_r[0]], rows_b[0], sem_g[0])

            def edge_quad(gq, _):
                for u in range(NSLOT):
                    j = gq * NSLOT + u
                    b = u
                    bn = (u + 1) % NSLOT

                    @pl.when(j + 1 < CPT)
                    def _():
                        load_pk(j + 1, bn)
                    pltpu.make_async_copy(
                        Yab.at[idx_r[b]], rows_b[b], sem_g[b]).wait()
                    pltpu.async_copy(rows_b[b], g.at[idx_c[b]], sem_s[b],
                                     add=True)
                    pltpu.make_async_copy(
                        rows_b[b], g.at[idx_c[b]], sem_s[b]).wait()

                    @pl.when(j + 1 < CPT)
                    def _():
                        prep(j + 1, bn)
                        pltpu.async_copy(
                            Yab.at[idx_r[bn]], rows_b[bn], sem_g[bn])
                return 0
            lax.fori_loop(0, CPT // NSLOT, edge_quad, 0)
            plsc.subcore_barrier()

            def resblk(bidx, _):
                base = bidx * RB
                pltpu.sync_copy(g.at[pl.ds(nbase + base, RB), :], rows_b[0])
                pltpu.sync_copy(
                    Yab.at[pl.ds(src_off + nbase + base, RB), :], aux_buf
                )

                def srow(r16, _):
                    sv16 = t_v[pl.ds(base + r16 * 16, 16)]
                    for ri in range(16):
                        sv = jnp.full((16,), sv16[ri], jnp.float32)
                        row = r16 * 16 + ri
                        for j in range(DH // 16):
                            sl = pl.ds(j * 16, 16)
                            rows_b[0][row, sl] = sv * (
                                rows_b[0][row, sl] + aux_buf[row, sl]
                            )
                    return 0
                lax.fori_loop(0, RB // 16, srow, 0)
                pltpu.sync_copy(
                    rows_b[0],
                    Yab.at[pl.ds(src_off + NP + nbase + base, RB), :],
                )
                pltpu.sync_copy(zbuf, g.at[pl.ds(nbase + base, RB), :])
                return 0
            lax.fori_loop(0, NB, resblk, 0)
            plsc.subcore_barrier()
            return 0
        lax.fori_loop(0, K, hop_body, 0)

    return k(fab, pk_edges)


def _tc_linear(YA, YB, WA9, WB9, sinv_col, b):
    nb = NP // 512

    def mm(ya_ref, yb_ref, wa_ref, wb_ref, sv_ref, b_ref, o_ref):
        kk = pl.program_id(1)

        @pl.when(kk == 0)
        def _():
            o_ref[...] = jnp.zeros_like(o_ref)

        o_ref[...] += lax.dot_general(
            ya_ref[0], wa_ref[0], (((1,), (1,)), ((), ())),
            preferred_element_type=jnp.float32,
        ) + lax.dot_general(
            yb_ref[0], wb_ref[0], (((1,), (1,)), ((), ())),
            preferred_element_type=jnp.float32,
        )

        @pl.when(kk == K)
        def _():
            o_ref[...] = o_ref[...] * sv_ref[...] + b_ref[...]

    return pl.pallas_call(
        mm,
        grid=(nb, K + 1),
        in_specs=[
            pl.BlockSpec((1, 512, DH), lambda i, kq: (kq, i, 0)),
            pl.BlockSpec((1, 512, DH), lambda i, kq: (kq, i, 0)),
            pl.BlockSpec((1, OUT_DIM, DH), lambda i, kq: (kq, 0, 0)),
            pl.BlockSpec((1, OUT_DIM, DH), lambda i, kq: (kq, 0, 0)),
            pl.BlockSpec((512, 1), lambda i, kq: (i, 0)),
            pl.BlockSpec((1, OUT_DIM), lambda i, kq: (0, 0)),
        ],
        out_specs=pl.BlockSpec((512, OUT_DIM), lambda i, kq: (i, 0)),
        out_shape=jax.ShapeDtypeStruct((NP, OUT_DIM), jnp.float32),
    )(YA, YB, WA9, WB9, sinv_col, b)


@jax.jit
def kernel(feature, edge_index, W, b):
    fa = jnp.pad(feature[:, :DH], ((0, NP - N), (0, 0)))
    fb = jnp.pad(feature[:, DH:], ((0, NP - N), (0, 0)))
    fab = jnp.concatenate([fa, fb], axis=0)
    row = edge_index[0]
    col = edge_index[1]
    row_p = jnp.pad(row, (0, EP - E))
    col_p = jnp.pad(col, (0, EP - E), constant_values=TRASH)
    pk_edges = jnp.stack(
        [row_p.reshape(NCHUNK, CH), col_p.reshape(NCHUNK, CH)], axis=1
    ).reshape(-1)

    Yab, sinv = _sc_propagate(fab, pk_edges)
    YA = Yab[:YROWS].reshape(K + 1, NP, DH)
    YB = Yab[YROWS:].reshape(K + 1, NP, DH)

    W9 = jnp.transpose(W.reshape(OUT_DIM, K + 1, D), (1, 0, 2))
    out = _tc_linear(YA, YB, W9[:, :, :DH], W9[:, :, DH:],
                     sinv[:NP].reshape(NP, 1), b.reshape(1, OUT_DIM))
    return out[:N]


# speedup vs baseline: 1.1445x; 1.1445x over previous
import functools

import jax
import jax.numpy as jnp
from jax import lax
from jax.experimental import pallas as pl
from jax.experimental.pallas import tpu as pltpu
from jax.experimental.pallas import tpu_sc as plsc

N = 10000
E = 320000
D = 128
K = 8
OUT_DIM = 128

DH = D // 2
NS = 16
NP = 10240
SLAB = NP // NS
CH = 128
NCHUNK = 2560
EP = NCHUNK * CH
CPT = NCHUNK // NS
TRASH = NP + 64
GROWS = NP + 256
RB = 128
NB = SLAB // RB
YROWS = (K + 1) * NP
NSLOT = 4


def _rsqrt16(d):
    bi = lax.bitcast_convert_type(d, jnp.int32)
    bi = jnp.int32(0x5F3759DF) - lax.shift_right_logical(bi, 1)
    x = lax.bitcast_convert_type(bi, jnp.float32)
    half = d * 0.5
    for _ in range(3):
        x = x * (1.5 - half * x * x)
    return x


def _sc_propagate(fab, pk_edges):
    mesh = plsc.VectorSubcoreMesh(
        core_axis_name="c", subcore_axis_name="s", num_cores=2
    )

    @functools.partial(
        pl.kernel,
        out_type=[
            jax.ShapeDtypeStruct((2 * YROWS, DH), jnp.float32),
            jax.ShapeDtypeStruct((2 * NP,), jnp.float32),
        ],
        mesh=mesh,
        compiler_params=pltpu.CompilerParams(use_tc_tiling_on_sc=False),
        scratch_types=[
            [pltpu.VMEM((CH, DH), jnp.float32) for _ in range(NSLOT)],
            [pltpu.VMEM((2 * CH,), jnp.int32) for _ in range(NSLOT)],
            [pltpu.VMEM((CH,), jnp.int32) for _ in range(NSLOT)],
            [pltpu.VMEM((CH,), jnp.int32) for _ in range(NSLOT)],
            pltpu.VMEM((CH, DH), jnp.float32),
            pltpu.VMEM((CH, DH), jnp.float32),
            pltpu.VMEM((CH,), jnp.float32),
            pltpu.VMEM((SLAB,), jnp.float32),
            pltpu.VMEM((SLAB,), jnp.float32),
            pltpu.VMEM((SLAB,), jnp.float32),
            pltpu.VMEM_SHARED((GROWS, DH), jnp.float32),
            pltpu.VMEM_SHARED((GROWS,), jnp.float32),
            [pltpu.SemaphoreType.DMA for _ in range(NSLOT)],
            [pltpu.SemaphoreType.DMA for _ in range(NSLOT)],
            [pltpu.SemaphoreType.DMA for _ in range(NSLOT)],
            [pltpu.SemaphoreType.DMA for _ in range(NSLOT)],
        ],
    )
    def k(fab_r, pk, Yab, sinv, rows_b, pk_b, idx_r, idx_c, aux_buf, zbuf,
          ones_buf, deg_v, s_v, t_v, g, deg1, sem_i, sem_c, sem_g, sem_s):
        cid = lax.axis_index("c")
        wid = lax.axis_index("s")
        nbase = wid * SLAB
        ybase = cid * YROWS

        def fill16(r, _):
            ones_buf[pl.ds(r * 16, 16)] = jnp.full((16,), 1.0, jnp.float32)
            return 0
        lax.fori_loop(0, CH // 16, fill16, 0)

        def fillz(r, _):
            for j in range(DH // 16):
                zbuf[r, pl.ds(j * 16, 16)] = jnp.zeros((16,), jnp.float32)
            return 0
        lax.fori_loop(0, CH, fillz, 0)

        def fill_slab(r, _):
            s_v[pl.ds(r * 16, 16)] = jnp.full((16,), 1.0, jnp.float32)
            return 0
        lax.fori_loop(0, SLAB // 16, fill_slab, 0)
        pltpu.sync_copy(s_v, deg1.at[pl.ds(nbase, SLAB)])

        def zeroblk(bidx, _):
            pltpu.sync_copy(zbuf, g.at[pl.ds(nbase + bidx * RB, RB), :])
            return 0
        lax.fori_loop(0, NB, zeroblk, 0)
        plsc.subcore_barrier()

        def hist(gq, _):
            for b in range(2):
                j = 2 * gq + b

                @pl.when(gq >= 1)
                def _():
                    pltpu.make_async_copy(
                        ones_buf, deg1.at[idx_c[b]], sem_s[b]
                    ).wait()
                off = (wid + NS * j) * 2 * CH + CH
                pltpu.sync_copy(pk.at[pl.ds(off, CH)], idx_c[b])
                pltpu.async_copy(ones_buf, deg1.at[idx_c[b]], sem_s[b],
                                 add=True)
            return 0
        lax.fori_loop(0, CPT // 2, hist, 0)
        for b in range(2):
            pltpu.make_async_copy(ones_buf, deg1.at[idx_c[b]], sem_s[b]).wait()
        plsc.subcore_barrier()

        pltpu.sync_copy(deg1.at[pl.ds(nbase, SLAB)], deg_v)

        def newton(r, _):
            sl = pl.ds(r * 16, 16)
            d = deg_v[sl]
            x = _rsqrt16(d)
            s_v[sl] = x
            t_v[sl] = x * x
            deg_v[sl] = d * x
            return 0
        lax.fori_loop(0, SLAB // 16, newton, 0)
        pltpu.sync_copy(deg_v, sinv.at[pl.ds(cid * NP + nbase, SLAB)])

        def y0blk(bidx, _):
            base = bidx * RB
            pltpu.sync_copy(
                fab_r.at[pl.ds(cid * NP + nbase + base, RB), :], rows_b[0]
            )

            def srow(r16, _):
                sv16 = s_v[pl.ds(base + r16 * 16, 16)]
                for ri in range(16):
                    sv = jnp.full((16,), sv16[ri], jnp.float32)
                    row = r16 * 16 + ri
                    for j in range(DH // 16):
                        sl = pl.ds(j * 16, 16)
                        rows_b[0][row, sl] = rows_b[0][row, sl] * sv
                return 0
            lax.fori_loop(0, RB // 16, srow, 0)
            pltpu.sync_copy(
                rows_b[0], Yab.at[pl.ds(ybase + nbase + base, RB), :]
            )
            return 0
        lax.fori_loop(0, NB, y0blk, 0)
        plsc.subcore_barrier()

        def hop_body(h, _):
            src_off = ybase + h * NP
            ovec = jnp.full((16,), src_off, jnp.int32)

            def prep(jj, b):
                pltpu.make_async_copy(
                    pk.at[pl.ds(0, 2 * CH)], pk_b[b], sem_i[b]).wait()
                pltpu.make_async_copy(
                    pk.at[pl.ds(0, CH)], idx_c[b], sem_c[b]).wait()

                def unpack(q, _):
                    sl = pl.ds(q * 16, 16)
                    idx_r[b][sl] = pk_b[b][sl] + ovec
                    return 0
                lax.fori_loop(0, CH // 16, unpack, 0)

            def load_pk(jj, b):
                off = (wid + NS * jj) * 2 * CH
                pltpu.async_copy(pk.at[pl.ds(off, 2 * CH)], pk_b[b], sem_i[b])
                pltpu.async_copy(pk.at[pl.ds(off + CH, CH)], idx_c[b],
                                 sem_c[b])

            load_pk(0, 0)
            prep(0, 0)
            pltpu.async_copy(Yab.at[idx_r[0]], rows_b[0], sem_g[0])

            def edge_quad(gq, _):
                for u in range(NSLOT):
                    j = gq * NSLOT + u
                    b = u
                    bn = (u + 1) % NSLOT

                    @pl.when(j >= NSLOT - 1)
                    def _():
                        pltpu.make_async_copy(
                            rows_b[bn], g.at[idx_c[bn]], sem_s[bn]).wait()

                    @pl.when(j + 1 < CPT)
                    def _():
                        load_pk(j + 1, bn)
                    pltpu.make_async_copy(
                        Yab.at[idx_r[b]], rows_b[b], sem_g[b]).wait()
                    pltpu.async_copy(rows_b[b], g.at[idx_c[b]], sem_s[b],
                                     add=True)

                    @pl.when(j + 1 < CPT)
                    def _():
                        prep(j + 1, bn)
                        pltpu.async_copy(
                            Yab.at[idx_r[bn]], rows_b[bn], sem_g[bn])
                return 0
            lax.fori_loop(0, CPT // NSLOT, edge_quad, 0)
            for b in [(CPT - 3) % NSLOT, (CPT - 2) % NSLOT, (CPT - 1) % NSLOT]:
                pltpu.make_async_copy(rows_b[b], g.at[idx_c[b]],
                                      sem_s[b]).wait()
            plsc.subcore_barrier()

            def resblk(bidx, _):
                base = bidx * RB
                pltpu.sync_copy(g.at[pl.ds(nbase + base, RB), :], rows_b[0])
                pltpu.sync_copy(
                    Yab.at[pl.ds(src_off + nbase + base, RB), :], aux_buf
                )

                def srow(r16, _):
                    sv16 = t_v[pl.ds(base + r16 * 16, 16)]
                    for ri in range(16):
                        sv = jnp.full((16,), sv16[ri], jnp.float32)
                        row = r16 * 16 + ri
                        for j in range(DH // 16):
                            sl = pl.ds(j * 16, 16)
                            rows_b[0][row, sl] = sv * (
                                rows_b[0][row, sl] + aux_buf[row, sl]
                            )
                    return 0
                lax.fori_loop(0, RB // 16, srow, 0)
                pltpu.sync_copy(
                    rows_b[0],
                    Yab.at[pl.ds(src_off + NP + nbase + base, RB), :],
                )
                pltpu.sync_copy(zbuf, g.at[pl.ds(nbase + base, RB), :])
                return 0
            lax.fori_loop(0, NB, resblk, 0)
            plsc.subcore_barrier()
            return 0
        lax.fori_loop(0, K, hop_body, 0)

    return k(fab, pk_edges)


def _tc_linear(YA, YB, WA9, WB9, sinv_col, b):
    nb = NP // 512

    def mm(ya_ref, yb_ref, wa_ref, wb_ref, sv_ref, b_ref, o_ref):
        kk = pl.program_id(1)

        @pl.when(kk == 0)
        def _():
            o_ref[...] = jnp.zeros_like(o_ref)

        o_ref[...] += lax.dot_general(
            ya_ref[0], wa_ref[0], (((1,), (1,)), ((), ())),
            preferred_element_type=jnp.float32,
        ) + lax.dot_general(
            yb_ref[0], wb_ref[0], (((1,), (1,)), ((), ())),
            preferred_element_type=jnp.float32,
        )

        @pl.when(kk == K)
        def _():
            o_ref[...] = o_ref[...] * sv_ref[...] + b_ref[...]

    return pl.pallas_call(
        mm,
        grid=(nb, K + 1),
        in_specs=[
            pl.BlockSpec((1, 512, DH), lambda i, kq: (kq, i, 0)),
            pl.BlockSpec((1, 512, DH), lambda i, kq: (kq, i, 0)),
            pl.BlockSpec((1, OUT_DIM, DH), lambda i, kq: (kq, 0, 0)),
            pl.BlockSpec((1, OUT_DIM, DH), lambda i, kq: (kq, 0, 0)),
            pl.BlockSpec((512, 1), lambda i, kq: (i, 0)),
            pl.BlockSpec((1, OUT_DIM), lambda i, kq: (0, 0)),
        ],
        out_specs=pl.BlockSpec((512, OUT_DIM), lambda i, kq: (i, 0)),
        out_shape=jax.ShapeDtypeStruct((NP, OUT_DIM), jnp.float32),
    )(YA, YB, WA9, WB9, sinv_col, b)


@jax.jit
def kernel(feature, edge_index, W, b):
    fa = jnp.pad(feature[:, :DH], ((0, NP - N), (0, 0)))
    fb = jnp.pad(feature[:, DH:], ((0, NP - N), (0, 0)))
    fab = jnp.concatenate([fa, fb], axis=0)
    row = edge_index[0]
    col = edge_index[1]
    row_p = jnp.pad(row, (0, EP - E))
    col_p = jnp.pad(col, (0, EP - E), constant_values=TRASH)
    pk_edges = jnp.stack(
        [row_p.reshape(NCHUNK, CH), col_p.reshape(NCHUNK, CH)], axis=1
    ).reshape(-1)

    Yab, sinv = _sc_propagate(fab, pk_edges)
    YA = Yab[:YROWS].reshape(K + 1, NP, DH)
    YB = Yab[YROWS:].reshape(K + 1, NP, DH)

    W9 = jnp.transpose(W.reshape(OUT_DIM, K + 1, D), (1, 0, 2))
    out = _tc_linear(YA, YB, W9[:, :, :DH], W9[:, :, DH:],
                     sinv[:NP].reshape(NP, 1), b.reshape(1, OUT_DIM))
    return out[:N]
